# Initial kernel scaffold; baseline (speedup 1.0000x reference)
import jax
import jax.numpy as jnp
from jax.experimental import pallas as pl


def kernel(o_wh, t_mask, t_ind, t_wh):
    raise NotImplementedError("write your pallas kernel here")


# speedup vs baseline: 1.1972x; 1.1972x over previous
import functools

import jax
import jax.numpy as jnp
from jax import lax
from jax.experimental import pallas as pl
from jax.experimental.pallas import tpu as pltpu
from jax.experimental.pallas import tpu_sc as plsc

B, C, H, W, K = 32, 2, 256, 256, 500
HW = H * W
KP = 512
NCHUNK = KP // 16
NGATHER = (C * KP) // 128
BPT = B // 16

_mesh = plsc.VectorSubcoreMesh(core_axis_name="c", subcore_axis_name="s")


@functools.partial(
    pl.kernel,
    mesh=_mesh,
    compiler_params=pltpu.CompilerParams(needs_layout_passes=False),
    out_type=jax.ShapeDtypeStruct((16,), jnp.float32),
    scratch_types=[
        pltpu.VMEM((KP,), jnp.int32),
        pltpu.VMEM((NGATHER, 128), jnp.int32),
        pltpu.VMEM((NGATHER, 128), jnp.float32),
        pltpu.VMEM((KP,), jnp.float32),
        pltpu.VMEM((C, KP), jnp.float32),
        pltpu.VMEM((32,), jnp.float32),
        pltpu.VMEM((16, 32), jnp.float32),
        pltpu.VMEM((16,), jnp.float32),
        pltpu.VMEM((16,), jnp.float32),
        pltpu.VMEM_SHARED((16, 32), jnp.float32),
        pltpu.SemaphoreType.DMA,
    ],
)
def _regl1_kernel(o_flat, t_ind, t_mask, t_wh, out, idx_v, addr_v, vals_v,
                  mask_v, twh_v, pbuf, allp, outst, tmp, shared, sem):
    cid = lax.axis_index("c")
    sid = lax.axis_index("s")

    @pl.when(cid == 0)
    def _work():
        acc = jnp.zeros((16,), jnp.float32)
        macc = jnp.zeros((16,), jnp.float32)
        for u in range(BPT):
            b = sid * BPT + u
            pltpu.sync_copy(t_ind.at[b], idx_v)
            pltpu.sync_copy(t_mask.at[b], mask_v)
            pltpu.sync_copy(t_wh.at[b], twh_v)
            base = b * (C * HW)
            for i in range(NCHUNK):
                iv = idx_v[pl.ds(i * 16, 16)] + base
                j0 = i * 16
                addr_v[j0 // 128, pl.ds(j0 % 128, 16)] = iv
                j1 = KP + i * 16
                addr_v[j1 // 128, pl.ds(j1 % 128, 16)] = iv + HW
            copies = [
                pltpu.async_copy(o_flat.at[addr_v.at[g]], vals_v.at[g], sem)
                for g in range(NGATHER)
            ]
            for cp in copies:
                cp.wait()
            for i in range(NCHUNK):
                m = mask_v[pl.ds(i * 16, 16)]
                macc = macc + m
                for c in range(C):
                    j = c * KP + i * 16
                    v = vals_v[j // 128, pl.ds(j % 128, 16)]
                    t = twh_v[c, pl.ds(i * 16, 16)]
                    acc = acc + jnp.abs(v * m - t * m)
        pbuf[pl.ds(0, 16)] = acc
        pbuf[pl.ds(16, 16)] = macc
        pltpu.sync_copy(pbuf, shared.at[sid])

    plsc.subcore_barrier()

    @pl.when((cid == 0) & (sid == 0))
    def _finish():
        pltpu.sync_copy(shared, allp)
        l1 = jnp.zeros((16,), jnp.float32)
        mm = jnp.zeros((16,), jnp.float32)
        for r in range(16):
            l1 = l1 + allp[r, pl.ds(0, 16)]
            mm = mm + allp[r, pl.ds(16, 16)]

        def lane_sum(x):
            for sh in (8, 4, 2, 1):
                tmp[...] = x
                perm = lax.iota(jnp.int32, 16) ^ sh
                x = x + plsc.load_gather(tmp, [perm])
            return x

        outst[...] = lane_sum(l1) / lane_sum(mm)
        pltpu.sync_copy(outst, out)


def kernel(o_wh, t_mask, t_ind, t_wh):
    o_flat = o_wh.reshape(B * C * HW)
    ind = jnp.zeros((B, KP), jnp.int32).at[:, :K].set(t_ind.astype(jnp.int32))
    mask = jnp.zeros((B, KP), jnp.float32).at[:, :K].set(t_mask)
    twh = jnp.zeros((B, C, KP), jnp.float32).at[:, :, :K].set(
        jnp.transpose(t_wh, (0, 2, 1)))
    out = _regl1_kernel(o_flat, ind, mask, twh)
    return out[0]


# speedup vs baseline: 1.2643x; 1.0560x over previous
import functools

import jax
import jax.numpy as jnp
from jax import lax
from jax.experimental import pallas as pl
from jax.experimental.pallas import tpu as pltpu
from jax.experimental.pallas import tpu_sc as plsc

B, C, H, W, K = 32, 2, 256, 256, 500
HW = H * W
KP = 512
NCHUNK = KP // 16
NW = 32

_mesh = plsc.VectorSubcoreMesh(core_axis_name="c", subcore_axis_name="s")


@functools.partial(
    pl.kernel,
    mesh=_mesh,
    compiler_params=pltpu.CompilerParams(needs_layout_passes=False),
    out_type=jax.ShapeDtypeStruct((NW, 32), jnp.float32),
    scratch_types=[
        pltpu.VMEM((KP,), jnp.int32),
        pltpu.VMEM((C * KP,), jnp.float32),
        pltpu.VMEM((KP,), jnp.float32),
        pltpu.VMEM((C, KP), jnp.float32),
        pltpu.VMEM((32,), jnp.float32),
        pltpu.SemaphoreType.DMA,
    ],
)
def _regl1_gather(o_flat, t_ind, t_mask, t_wh, out_part,
                  idx_v, vals_v, mask_v, twh_v, pbuf, sem):
    cid = lax.axis_index("c")
    sid = lax.axis_index("s")
    b = sid * 2 + cid

    pltpu.sync_copy(t_ind.at[b], idx_v)
    pltpu.sync_copy(t_mask.at[b], mask_v)
    pltpu.sync_copy(t_wh.at[b], twh_v)
    base = b * (C * HW)
    copies = []
    for i in range(NCHUNK):
        iv = idx_v[pl.ds(i * 16, 16)] + base
        copies.append(pltpu.async_copy(
            o_flat.at[iv], vals_v.at[pl.ds(i * 16, 16)], sem))
        copies.append(pltpu.async_copy(
            o_flat.at[iv + HW], vals_v.at[pl.ds(KP + i * 16, 16)], sem))
    for cp in copies:
        cp.wait()
    acc = jnp.zeros((16,), jnp.float32)
    macc = jnp.zeros((16,), jnp.float32)
    for i in range(NCHUNK):
        m = mask_v[pl.ds(i * 16, 16)]
        macc = macc + m
        for c in range(C):
            v = vals_v[pl.ds(c * KP + i * 16, 16)]
            t = twh_v[c, pl.ds(i * 16, 16)]
            acc = acc + jnp.abs(v * m - t * m)
    pbuf[pl.ds(0, 16)] = acc
    pbuf[pl.ds(16, 16)] = macc
    pltpu.sync_copy(pbuf, out_part.at[b])


def _combine_body(p_ref, o_ref):
    loss = jnp.sum(p_ref[:, :16]) / jnp.sum(p_ref[:, 16:])
    o_ref[...] = jnp.full((1, 1), loss, jnp.float32)


_combine = pl.pallas_call(
    _combine_body,
    out_shape=jax.ShapeDtypeStruct((1, 1), jnp.float32),
)


def kernel(o_wh, t_mask, t_ind, t_wh):
    o_flat = o_wh.reshape(B * C * HW)
    ind = jnp.zeros((B, KP), jnp.int32).at[:, :K].set(t_ind.astype(jnp.int32))
    mask = jnp.zeros((B, KP), jnp.float32).at[:, :K].set(t_mask)
    twh = jnp.zeros((B, C, KP), jnp.float32).at[:, :, :K].set(
        jnp.transpose(t_wh, (0, 2, 1)))
    part = _regl1_gather(o_flat, ind, mask, twh)
    return _combine(part)[0, 0]


# speedup vs baseline: 1.6187x; 1.2803x over previous
import functools

import jax
import jax.numpy as jnp
from jax import lax
from jax.experimental import pallas as pl
from jax.experimental.pallas import tpu as pltpu
from jax.experimental.pallas import tpu_sc as plsc

B, C, H, W, K = 32, 2, 256, 256, 500
HW = H * W
KP = 512
NCHUNK = KP // 16
NW = 32

_mesh = plsc.VectorSubcoreMesh(core_axis_name="c", subcore_axis_name="s")


@functools.partial(
    pl.kernel,
    mesh=_mesh,
    compiler_params=pltpu.CompilerParams(needs_layout_passes=False),
    out_type=jax.ShapeDtypeStruct((NW, 32), jnp.float32),
    scratch_types=[
        pltpu.VMEM((KP,), jnp.int32),
        pltpu.VMEM((H // 8, W // 128, 8, 128), jnp.float32),
        pltpu.VMEM((KP,), jnp.float32),
        pltpu.VMEM((C, KP), jnp.float32),
        pltpu.VMEM((32,), jnp.float32),
        pltpu.SemaphoreType.DMA,
    ],
)
def _regl1_gather(o_t, t_ind, t_mask, t_wh, out_part,
                  idx_v, plane_v, mask_v, twh_v, pbuf, sem):
    cid = lax.axis_index("c")
    sid = lax.axis_index("s")
    b = sid * 2 + cid

    pltpu.sync_copy(t_ind.at[b], idx_v)
    pltpu.sync_copy(t_mask.at[b], mask_v)
    pltpu.sync_copy(t_wh.at[b], twh_v)

    acc = jnp.zeros((16,), jnp.float32)
    macc = jnp.zeros((16,), jnp.float32)
    for c in range(C):
        pltpu.async_copy(o_t.at[b, c], plane_v, sem).wait()
        for i in range(NCHUNK):
            idx = idx_v[pl.ds(i * 16, 16)]
            m = mask_v[pl.ds(i * 16, 16)]
            if c == 0:
                macc = macc + m
            v = plsc.load_gather(
                plane_v,
                [idx >> 11, (idx >> 7) & 1, (idx >> 8) & 7, idx & 127])
            t = twh_v[c, pl.ds(i * 16, 16)]
            acc = acc + jnp.abs(v * m - t * m)
    pbuf[pl.ds(0, 16)] = acc
    pbuf[pl.ds(16, 16)] = macc
    pltpu.sync_copy(pbuf, out_part.at[b])


def _combine_body(p_ref, o_ref):
    loss = jnp.sum(p_ref[:, :16]) / jnp.sum(p_ref[:, 16:])
    o_ref[...] = jnp.full((1, 1), loss, jnp.float32)


_combine = pl.pallas_call(
    _combine_body,
    out_shape=jax.ShapeDtypeStruct((1, 1), jnp.float32),
)


def kernel(o_wh, t_mask, t_ind, t_wh):
    o_t = jnp.transpose(
        o_wh.reshape(B, C, H // 8, 8, W // 128, 128), (0, 1, 2, 4, 3, 5))
    ind = jnp.zeros((B, KP), jnp.int32).at[:, :K].set(t_ind.astype(jnp.int32))
    mask = jnp.zeros((B, KP), jnp.float32).at[:, :K].set(t_mask)
    twh = jnp.zeros((B, C, KP), jnp.float32).at[:, :, :K].set(
        jnp.transpose(t_wh, (0, 2, 1)))
    part = _regl1_gather(o_t, ind, mask, twh)
    return _combine(part)[0, 0]


# speedup vs baseline: 1.8406x; 1.1371x over previous
import functools

import jax
import jax.numpy as jnp
from jax import lax
from jax.experimental import pallas as pl
from jax.experimental.pallas import tpu as pltpu
from jax.experimental.pallas import tpu_sc as plsc

B, C, H, W, K = 32, 2, 256, 256, 500
HW = H * W
KP = 512
NCHUNK = KP // 16
NW = 32

_mesh = plsc.VectorSubcoreMesh(core_axis_name="c", subcore_axis_name="s")


@functools.partial(
    pl.kernel,
    mesh=_mesh,
    compiler_params=pltpu.CompilerParams(needs_layout_passes=False),
    out_type=jax.ShapeDtypeStruct((NW, 32), jnp.float32),
    scratch_types=[
        pltpu.VMEM((KP,), jnp.int32),
        pltpu.VMEM((C * KP,), jnp.float32),
        pltpu.VMEM((KP,), jnp.float32),
        pltpu.VMEM((C, KP), jnp.float32),
        pltpu.VMEM((32,), jnp.float32),
        pltpu.SemaphoreType.DMA,
    ],
)
def _regl1_gather(o_lin, t_ind, t_mask, t_wh, out_part,
                  idx_v, vals_v, mask_v, twh_v, pbuf, sem):
    cid = lax.axis_index("c")
    sid = lax.axis_index("s")
    b = sid * 2 + cid

    pltpu.sync_copy(t_ind.at[b], idx_v)
    pltpu.sync_copy(t_mask.at[b], mask_v)
    pltpu.sync_copy(t_wh.at[b], twh_v)

    copies = []
    for c in range(C):
        base = (b * C + c) * HW
        for i in range(NCHUNK):
            idx = idx_v[pl.ds(i * 16, 16)]
            paddr = (base
                     + (idx >> 11) * 2048
                     + ((idx >> 7) & 1) * 1024
                     + ((idx >> 8) & 7) * 128
                     + (idx & 127))
            copies.append(pltpu.async_copy(
                o_lin.at[paddr], vals_v.at[pl.ds(c * KP + i * 16, 16)], sem))
    for cp in copies:
        cp.wait()

    acc = jnp.zeros((16,), jnp.float32)
    macc = jnp.zeros((16,), jnp.float32)
    for i in range(NCHUNK):
        m = mask_v[pl.ds(i * 16, 16)]
        macc = macc + m
        for c in range(C):
            v = vals_v[pl.ds(c * KP + i * 16, 16)]
            t = twh_v[c, pl.ds(i * 16, 16)]
            acc = acc + jnp.abs(v * m - t * m)
    pbuf[pl.ds(0, 16)] = acc
    pbuf[pl.ds(16, 16)] = macc
    pltpu.sync_copy(pbuf, out_part.at[b])


def _combine_body(p_ref, o_ref):
    loss = jnp.sum(p_ref[:, :16]) / jnp.sum(p_ref[:, 16:])
    o_ref[...] = jnp.full((1, 1), loss, jnp.float32)


_combine = pl.pallas_call(
    _combine_body,
    out_shape=jax.ShapeDtypeStruct((1, 1), jnp.float32),
)


def kernel(o_wh, t_mask, t_ind, t_wh):
    o_lin = jnp.transpose(
        o_wh.reshape(B, C, H // 8, 8, W // 128, 128),
        (0, 1, 2, 4, 3, 5)).reshape(B * C * HW)
    ind = jnp.zeros((B, KP), jnp.int32).at[:, :K].set(t_ind.astype(jnp.int32))
    mask = jnp.zeros((B, KP), jnp.float32).at[:, :K].set(t_mask)
    twh = jnp.zeros((B, C, KP), jnp.float32).at[:, :, :K].set(
        jnp.transpose(t_wh, (0, 2, 1)))
    part = _regl1_gather(o_lin, ind, mask, twh)
    return _combine(part)[0, 0]
